# baseline (speedup 1.0000x reference)
import functools

import jax
import jax.numpy as jnp
from jax import lax
from jax.experimental import pallas as pl
from jax.experimental.pallas import tpu as pltpu
from jax.experimental.pallas import tpu_sc as plsc

B, L, W_CHARS = 1024, 50, 16
CHAR_SIZE = 128
CHAR_DIM = 30
CD_PAD = 32
HIDDEN = 1024
N_TOK = B * L
N_LOOK = N_TOK * W_CHARS
K_PAD = W_CHARS * CD_PAD

_NC, _NS = 2, 16
_NW = _NC * _NS
_IDX_W = 128
_RPW = N_LOOK // _NW // _IDX_W
_KF = 10
_NG = _RPW // _KF
_GLOOK = _KF * _IDX_W

_sc_mesh = plsc.VectorSubcoreMesh(
    core_axis_name="c", subcore_axis_name="s", num_cores=_NC, num_subcores=_NS
)


@functools.partial(
    pl.kernel,
    out_type=jax.ShapeDtypeStruct((N_LOOK, CD_PAD), jnp.float32),
    mesh=_sc_mesh,
    scratch_types=[
        pltpu.VMEM((_RPW, _IDX_W), jnp.int32),
        pltpu.VMEM((2, _GLOOK, CD_PAD), jnp.float32),
        pltpu.SemaphoreType.DMA,
        pltpu.SemaphoreType.DMA,
    ],
    compiler_params=pltpu.CompilerParams(use_tc_tiling_on_sc=False),
)
def _sc_gather(idx_hbm, tab_hbm, emb_hbm, idx_v, rows_v, gsem, wsem):
    wid = lax.axis_index("s") * _NC + lax.axis_index("c")
    pltpu.sync_copy(idx_hbm.at[wid], idx_v)
    base = wid * (_RPW * _IDX_W)

    def write_desc(g, b):
        return pltpu.make_async_copy(
            rows_v.at[b],
            emb_hbm.at[pl.ds(base + g * _GLOOK, _GLOOK)],
            wsem,
        )

    def fire_gathers(g, b):
        return [
            pltpu.async_copy(
                tab_hbm.at[idx_v.at[g * _KF + j]],
                rows_v.at[b].at[pl.ds(j * _IDX_W, _IDX_W)],
                gsem,
            )
            for j in range(_KF)
        ]

    @pl.loop(0, _NG, step=2)
    def _group(g0):
        for nb in range(2):
            g = g0 + nb

            @pl.when(g >= 2)
            def _():
                write_desc(g - 2, nb).wait()

            cps = fire_gathers(g, nb)
            for c in cps:
                c.wait()
            write_desc(g, nb).start()

    for nb in range(2):
        write_desc(_NG - 2 + nb, nb).wait()


_TB = 512


def _mm_body(e_ref, wt_ref, o_ref):
    o_ref[:] = jnp.dot(
        e_ref[:].astype(jnp.bfloat16), wt_ref[:], preferred_element_type=jnp.float32
    )


@jax.jit
def kernel(X, char_table, W_proj):
    idx = X.reshape(_NW, _RPW, _IDX_W)
    tab32 = jnp.pad(char_table, ((0, 0), (0, CD_PAD - CHAR_DIM)))
    emb = _sc_gather(idx, tab32)

    wt = jnp.pad(
        W_proj.reshape(HIDDEN, W_CHARS, CHAR_DIM),
        ((0, 0), (0, 0), (0, CD_PAD - CHAR_DIM)),
    ).reshape(HIDDEN, K_PAD).T.astype(jnp.bfloat16)

    out = pl.pallas_call(
        _mm_body,
        grid=(N_TOK // _TB,),
        in_specs=[
            pl.BlockSpec((_TB, K_PAD), lambda i: (i, 0)),
            pl.BlockSpec((K_PAD, HIDDEN), lambda i: (0, 0)),
        ],
        out_specs=pl.BlockSpec((_TB, HIDDEN), lambda i: (i, 0)),
        out_shape=jax.ShapeDtypeStruct((N_TOK, HIDDEN), jnp.float32),
    )(emb.reshape(N_TOK, K_PAD), wt)
    return out.reshape(B, L, HIDDEN)
